# Initial kernel scaffold; baseline (speedup 1.0000x reference)
import jax
import jax.numpy as jnp
from jax.experimental import pallas as pl


def kernel(inputs_embeds, pos_table, ln_gamma, ln_beta):
    raise NotImplementedError("write your pallas kernel here")


# speedup vs baseline: 2.0698x; 2.0698x over previous
import jax
import jax.numpy as jnp
from jax.experimental import pallas as pl

SEQ_LEN = 8192
D = 768
B = 4
EPS = 1e-12

BLOCK_ROWS = 1024


def _fused_ln_kernel(x_ref, p_ref, g_ref, b_ref, o_ref):
    x = x_ref[...] + p_ref[...]
    mean = jnp.mean(x, axis=-1, keepdims=True)
    xc = x - mean
    var = jnp.mean(xc * xc, axis=-1, keepdims=True)
    o_ref[...] = xc * jax.lax.rsqrt(var + EPS) * g_ref[...] + b_ref[...]


def kernel(inputs_embeds, pos_table, ln_gamma, ln_beta):
    b, s, d = inputs_embeds.shape
    x = inputs_embeds.reshape(b * s, d)
    g = ln_gamma.reshape(1, d)
    bt = ln_beta.reshape(1, d)
    n_rows = b * s
    grid = (n_rows // BLOCK_ROWS,)
    pos_blocks_per_seq = s // BLOCK_ROWS

    out = pl.pallas_call(
        _fused_ln_kernel,
        grid=grid,
        in_specs=[
            pl.BlockSpec((BLOCK_ROWS, d), lambda i: (i, 0)),
            pl.BlockSpec((BLOCK_ROWS, d), lambda i: (i % pos_blocks_per_seq, 0)),
            pl.BlockSpec((1, d), lambda i: (0, 0)),
            pl.BlockSpec((1, d), lambda i: (0, 0)),
        ],
        out_specs=pl.BlockSpec((BLOCK_ROWS, d), lambda i: (i, 0)),
        out_shape=jax.ShapeDtypeStruct((n_rows, d), inputs_embeds.dtype),
    )(x, pos_table, g, bt)
    return out.reshape(b, s, d)


# speedup vs baseline: 2.3581x; 1.1393x over previous
import jax
import jax.numpy as jnp
from jax.experimental import pallas as pl

SEQ_LEN = 8192
D = 768
B = 4
EPS = 1e-12

BLOCK_ROWS = 1024


def _fused_ln_kernel(x_ref, p_ref, g_ref, b_ref, o_ref):
    x = x_ref[...] + p_ref[...]
    mean = jnp.mean(x, axis=-1, keepdims=True)
    xc = x - mean
    var = jnp.mean(xc * xc, axis=-1, keepdims=True)
    o_ref[...] = xc * jax.lax.rsqrt(var + EPS) * g_ref[...] + b_ref[...]


def kernel(inputs_embeds, pos_table, ln_gamma, ln_beta):
    b, s, d = inputs_embeds.shape
    g = ln_gamma.reshape(1, d)
    bt = ln_beta.reshape(1, d)
    grid = (s // BLOCK_ROWS, b)

    out = pl.pallas_call(
        _fused_ln_kernel,
        grid=grid,
        in_specs=[
            pl.BlockSpec((1, BLOCK_ROWS, d), lambda i, j: (j, i, 0)),
            pl.BlockSpec((BLOCK_ROWS, d), lambda i, j: (i, 0)),
            pl.BlockSpec((1, d), lambda i, j: (0, 0)),
            pl.BlockSpec((1, d), lambda i, j: (0, 0)),
        ],
        out_specs=pl.BlockSpec((1, BLOCK_ROWS, d), lambda i, j: (j, i, 0)),
        out_shape=jax.ShapeDtypeStruct((b, s, d), inputs_embeds.dtype),
    )(inputs_embeds, pos_table, g, bt)
    return out


# speedup vs baseline: 2.6263x; 1.1138x over previous
import jax
import jax.numpy as jnp
from jax.experimental import pallas as pl

SEQ_LEN = 8192
D = 768
B = 4
EPS = 1e-12

BLOCK_ROWS = 2048


def _fused_ln_kernel(x_ref, p_ref, g_ref, b_ref, o_ref):
    x = x_ref[...] + p_ref[...]
    mean = jnp.mean(x, axis=-1, keepdims=True)
    xc = x - mean
    var = jnp.mean(xc * xc, axis=-1, keepdims=True)
    o_ref[...] = xc * jax.lax.rsqrt(var + EPS) * g_ref[...] + b_ref[...]


def kernel(inputs_embeds, pos_table, ln_gamma, ln_beta):
    b, s, d = inputs_embeds.shape
    g = ln_gamma.reshape(1, d)
    bt = ln_beta.reshape(1, d)
    grid = (s // BLOCK_ROWS, b)

    out = pl.pallas_call(
        _fused_ln_kernel,
        grid=grid,
        in_specs=[
            pl.BlockSpec((1, BLOCK_ROWS, d), lambda i, j: (j, i, 0)),
            pl.BlockSpec((BLOCK_ROWS, d), lambda i, j: (i, 0)),
            pl.BlockSpec((1, d), lambda i, j: (0, 0)),
            pl.BlockSpec((1, d), lambda i, j: (0, 0)),
        ],
        out_specs=pl.BlockSpec((1, BLOCK_ROWS, d), lambda i, j: (j, i, 0)),
        out_shape=jax.ShapeDtypeStruct((b, s, d), inputs_embeds.dtype),
    )(inputs_embeds, pos_table, g, bt)
    return out
